# Initial kernel scaffold; baseline (speedup 1.0000x reference)
import jax
import jax.numpy as jnp
from jax.experimental import pallas as pl


def kernel(edge_index, features, W1, b1, W2, b2):
    raise NotImplementedError("write your pallas kernel here")


# speedup vs baseline: 4.5339x; 4.5339x over previous
import functools

import jax
import jax.numpy as jnp
from jax import lax
from jax.experimental import pallas as pl
from jax.experimental.pallas import tpu as pltpu
from jax.experimental.pallas import tpu_sc as plsc

NC = 2
NS = 16
CHUNK = 128


def _seg_body(do_gather, n, e, width, *refs):
    if do_gather:
        (tbl, srci, dsti, zer, out, idx_s, idx_d, rows, acc, gsem) = refs
    else:
        (dsti, zer, ones, out, idx_d, rows, acc, gsem) = refs

    c = lax.axis_index("c")
    s = lax.axis_index("s")
    w = s * NC + c

    rpt = n // NS
    rpt_al = ((rpt + 7) // 8) * 8
    r0 = pl.multiple_of((s * rpt) // 8 * 8, 8)
    off = 0
    while off < rpt_al:
        ln = min(CHUNK, rpt_al - off)
        assert ln % 8 == 0
        pltpu.sync_copy(zer.at[pl.ds(0, ln)], acc.at[pl.ds(r0 + off, ln)])
        off += ln
    if not do_gather:
        pltpu.sync_copy(ones, rows)
    plsc.subcore_barrier()

    nch_total = e // CHUNK
    nch = (nch_total - w + (NC * NS) - 1) // (NC * NS)

    def step(j, _):
        base = (w + j * NC * NS) * CHUNK
        pltpu.sync_copy(dsti.at[pl.ds(base, CHUNK)], idx_d)
        if do_gather:
            pltpu.sync_copy(srci.at[pl.ds(base, CHUNK)], idx_s)
            pltpu.async_copy(tbl.at[idx_s], rows, gsem).wait()
        pltpu.sync_copy(rows, acc.at[idx_d], add=True)
        return 0

    lax.fori_loop(0, nch, step, 0)
    plsc.subcore_barrier()

    pltpu.sync_copy(acc.at[pl.ds(r0, rpt_al)],
                    out.at[pl.ds(c * n + r0, rpt_al)])


def _sc_segment_sum(table, src, dst, n):
    e = src.shape[0]
    width = table.shape[1]
    mesh = plsc.VectorSubcoreMesh(core_axis_name="c", subcore_axis_name="s",
                                  num_cores=NC, num_subcores=NS)
    body = functools.partial(_seg_body, True, n, e, width)
    fn = pl.kernel(
        body,
        out_type=jax.ShapeDtypeStruct((NC * n, width), jnp.float32),
        mesh=mesh,
        scratch_types=[
            pltpu.VMEM((CHUNK,), jnp.int32),
            pltpu.VMEM((CHUNK,), jnp.int32),
            pltpu.VMEM((CHUNK, width), jnp.float32),
            pltpu.VMEM_SHARED((n, width), jnp.float32),
            pltpu.SemaphoreType.DMA,
        ],
    )
    zer = jnp.zeros((CHUNK, width), jnp.float32)
    return fn(table, src, dst, zer).reshape(NC, n, width)


def _sc_degree(dst, n):
    e = dst.shape[0]
    width = 128
    mesh = plsc.VectorSubcoreMesh(core_axis_name="c", subcore_axis_name="s",
                                  num_cores=NC, num_subcores=NS)
    body = functools.partial(_seg_body, False, n, e, width)
    fn = pl.kernel(
        body,
        out_type=jax.ShapeDtypeStruct((NC * n, width), jnp.float32),
        mesh=mesh,
        scratch_types=[
            pltpu.VMEM((CHUNK,), jnp.int32),
            pltpu.VMEM((CHUNK, width), jnp.float32),
            pltpu.VMEM_SHARED((n, width), jnp.float32),
            pltpu.SemaphoreType.DMA,
        ],
    )
    zer = jnp.zeros((CHUNK, width), jnp.float32)
    ones = jnp.ones((CHUNK, width), jnp.float32)
    return fn(dst, zer, ones).reshape(NC, n, width)


def _tc_prep(deg_parts, features, W1):
    n, d = features.shape
    h = W1.shape[1]

    def body(p_ref, f_ref, w_ref, q_ref, nrm_ref):
        deg = p_ref[0, :, 0:1] + p_ref[1, :, 0:1]
        nrm = lax.rsqrt(jnp.maximum(deg, 1.0))
        nrm_ref[...] = nrm
        q_ref[...] = jnp.dot(f_ref[...], w_ref[...],
                             preferred_element_type=jnp.float32) * nrm

    return pl.pallas_call(
        body,
        out_shape=(jax.ShapeDtypeStruct((n, h), jnp.float32),
                   jax.ShapeDtypeStruct((n, 1), jnp.float32)),
    )(deg_parts, features, W1)


def _tc_mid(agg_parts, norm, b1):
    n, h = agg_parts.shape[1:]

    def body(p_ref, nrm_ref, b_ref, y_ref):
        a = (p_ref[0] + p_ref[1]) * nrm_ref[...] + b_ref[...][None, :]
        x = jnp.where(a > 0, a, jnp.exp(a) - 1.0)
        y_ref[...] = x * nrm_ref[...]

    return pl.pallas_call(
        body,
        out_shape=jax.ShapeDtypeStruct((n, h), jnp.float32),
    )(agg_parts, norm, b1)


def _tc_out(agg_parts, norm, W2, b2):
    n = norm.shape[0]
    c = b2.shape[0]

    def body(p_ref, nrm_ref, w_ref, b_ref, o_ref):
        a = (p_ref[0] + p_ref[1]) * nrm_ref[...]
        o_ref[...] = jnp.dot(a, w_ref[...],
                             preferred_element_type=jnp.float32) + b_ref[...][None, :]

    return pl.pallas_call(
        body,
        out_shape=jax.ShapeDtypeStruct((n, c), jnp.float32),
    )(agg_parts, norm, W2, b2)


def kernel(edge_index, features, W1, b1, W2, b2):
    n = features.shape[0]
    src = edge_index[0]
    dst = edge_index[1]

    deg_parts = _sc_degree(dst, n)
    q, norm = _tc_prep(deg_parts, features, W1)
    agg1 = _sc_segment_sum(q, src, dst, n)
    y = _tc_mid(agg1, norm, b1)
    agg2 = _sc_segment_sum(y, src, dst, n)
    return _tc_out(agg2, norm, W2, b2)


# speedup vs baseline: 8.0392x; 1.7731x over previous
import functools

import jax
import jax.numpy as jnp
from jax import lax
from jax.experimental import pallas as pl
from jax.experimental.pallas import tpu as pltpu
from jax.experimental.pallas import tpu_sc as plsc

NC = 2
NS = 16
CHUNK = 128


def _chunk_size(ept):
    for ch in range(128, 0, -8):
        if ept % ch == 0:
            return ch
    return 8


def _zero_acc(zer, acc, n, s):
    rpt = n // NS
    rpt_al = ((rpt + 7) // 8) * 8
    r0 = pl.multiple_of((s * rpt) // 8 * 8, 8)
    off = 0
    while off < rpt_al:
        ln = min(CHUNK, rpt_al - off)
        pltpu.sync_copy(zer.at[pl.ds(0, ln)], acc.at[pl.ds(r0 + off, ln)])
        off += ln
    return r0, rpt_al


def _agg_body(n, nch, *refs):
    (tbl, src2, dst3, zer, out, idxs, idxd, rows0, rows1, acc,
     sem0, sem1) = refs
    ch = rows0.shape[0]

    c = lax.axis_index("c")
    s = lax.axis_index("s")
    w = s * NC + c

    r0, rpt_al = _zero_acc(zer, acc, n, s)
    plsc.subcore_barrier()

    pltpu.sync_copy(src2.at[w], idxs)
    pltpu.sync_copy(dst3.at[w], idxd)

    def gather(j, buf, sem):
        return pltpu.async_copy(tbl.at[idxs.at[pl.ds(j * ch, ch)]], buf, sem)

    def drain(j, buf, sem):
        pltpu.make_async_copy(tbl.at[idxs.at[pl.ds(j * ch, ch)]], buf,
                              sem).wait()

    gather(0, rows0, sem0)

    def step(i, _):
        a = 2 * i
        gather(a + 1, rows1, sem1)
        drain(a, rows0, sem0)
        pltpu.sync_copy(rows0, acc.at[idxd.at[a]], add=True)
        gather(a + 2, rows0, sem0)
        drain(a + 1, rows1, sem1)
        pltpu.sync_copy(rows1, acc.at[idxd.at[a + 1]], add=True)
        return 0

    assert nch % 2 == 1, "pipeline epilogue assumes odd chunk count"
    lax.fori_loop(0, (nch - 1) // 2, step, 0)
    drain(nch - 1, rows0, sem0)
    pltpu.sync_copy(rows0, acc.at[idxd.at[nch - 1]], add=True)

    plsc.subcore_barrier()
    pltpu.sync_copy(acc.at[pl.ds(r0, rpt_al)],
                    out.at[pl.ds(c * n + r0, rpt_al)])


def _deg_body(n, nch, *refs):
    (dst3, zer, ones, out, idxd, rows, acc, gsem) = refs

    c = lax.axis_index("c")
    s = lax.axis_index("s")
    w = s * NC + c

    r0, rpt_al = _zero_acc(zer, acc, n, s)
    pltpu.sync_copy(ones, rows)
    plsc.subcore_barrier()

    pltpu.sync_copy(dst3.at[w], idxd)

    def step(j, _):
        pltpu.sync_copy(rows, acc.at[idxd.at[j]], add=True)
        return 0

    lax.fori_loop(0, nch, step, 0)
    plsc.subcore_barrier()
    pltpu.sync_copy(acc.at[pl.ds(r0, rpt_al)],
                    out.at[pl.ds(c * n + r0, rpt_al)])


def _sc_segment_sum(table, src, dst, n):
    e = src.shape[0]
    width = table.shape[1]
    ept = e // (NC * NS)
    ch = _chunk_size(ept)
    nch = ept // ch
    src2 = src.reshape(NC * NS, ept)
    dst3 = dst.reshape(NC * NS, nch, ch)
    mesh = plsc.VectorSubcoreMesh(core_axis_name="c", subcore_axis_name="s",
                                  num_cores=NC, num_subcores=NS)
    body = functools.partial(_agg_body, n, nch)
    fn = pl.kernel(
        body,
        out_type=jax.ShapeDtypeStruct((NC * n, width), jnp.float32),
        mesh=mesh,
        scratch_types=[
            pltpu.VMEM((ept,), jnp.int32),
            pltpu.VMEM((nch, ch), jnp.int32),
            pltpu.VMEM((ch, width), jnp.float32),
            pltpu.VMEM((ch, width), jnp.float32),
            pltpu.VMEM_SHARED((n, width), jnp.float32),
            pltpu.SemaphoreType.DMA,
            pltpu.SemaphoreType.DMA,
        ],
    )
    zer = jnp.zeros((CHUNK, width), jnp.float32)
    return fn(table, src2, dst3, zer).reshape(NC, n, width)


def _sc_degree(dst, n):
    e = dst.shape[0]
    width = 128
    ept = e // (NC * NS)
    ch = _chunk_size(ept)
    nch = ept // ch
    dst3 = dst.reshape(NC * NS, nch, ch)
    mesh = plsc.VectorSubcoreMesh(core_axis_name="c", subcore_axis_name="s",
                                  num_cores=NC, num_subcores=NS)
    body = functools.partial(_deg_body, n, nch)
    fn = pl.kernel(
        body,
        out_type=jax.ShapeDtypeStruct((NC * n, width), jnp.float32),
        mesh=mesh,
        scratch_types=[
            pltpu.VMEM((nch, ch), jnp.int32),
            pltpu.VMEM((ch, width), jnp.float32),
            pltpu.VMEM_SHARED((n, width), jnp.float32),
            pltpu.SemaphoreType.DMA,
        ],
    )
    zer = jnp.zeros((CHUNK, width), jnp.float32)
    ones = jnp.ones((ch, width), jnp.float32)
    return fn(dst3, zer, ones).reshape(NC, n, width)


def _tc_prep(deg_parts, features, W1):
    n, d = features.shape
    h = W1.shape[1]

    def body(p_ref, f_ref, w_ref, q_ref, nrm_ref):
        deg = p_ref[0, :, 0:1] + p_ref[1, :, 0:1]
        nrm = lax.rsqrt(jnp.maximum(deg, 1.0))
        nrm_ref[...] = nrm
        q_ref[...] = jnp.dot(f_ref[...], w_ref[...],
                             preferred_element_type=jnp.float32) * nrm

    return pl.pallas_call(
        body,
        out_shape=(jax.ShapeDtypeStruct((n, h), jnp.float32),
                   jax.ShapeDtypeStruct((n, 1), jnp.float32)),
    )(deg_parts, features, W1)


def _tc_mid(agg_parts, norm, b1):
    n, h = agg_parts.shape[1:]

    def body(p_ref, nrm_ref, b_ref, y_ref):
        a = (p_ref[0] + p_ref[1]) * nrm_ref[...] + b_ref[...][None, :]
        x = jnp.where(a > 0, a, jnp.exp(a) - 1.0)
        y_ref[...] = x * nrm_ref[...]

    return pl.pallas_call(
        body,
        out_shape=jax.ShapeDtypeStruct((n, h), jnp.float32),
    )(agg_parts, norm, b1)


def _tc_out(agg_parts, norm, W2, b2):
    n = norm.shape[0]
    c = b2.shape[0]

    def body(p_ref, nrm_ref, w_ref, b_ref, o_ref):
        a = (p_ref[0] + p_ref[1]) * nrm_ref[...]
        o_ref[...] = jnp.dot(a, w_ref[...],
                             preferred_element_type=jnp.float32) + b_ref[...][None, :]

    return pl.pallas_call(
        body,
        out_shape=jax.ShapeDtypeStruct((n, c), jnp.float32),
    )(agg_parts, norm, W2, b2)


def kernel(edge_index, features, W1, b1, W2, b2):
    n = features.shape[0]
    src = edge_index[0]
    dst = edge_index[1]

    deg_parts = _sc_degree(dst, n)
    q, norm = _tc_prep(deg_parts, features, W1)
    agg1 = _sc_segment_sum(q, src, dst, n)
    y = _tc_mid(agg1, norm, b1)
    agg2 = _sc_segment_sum(y, src, dst, n)
    return _tc_out(agg2, norm, W2, b2)


# speedup vs baseline: 8.0713x; 1.0040x over previous
import functools

import jax
import jax.numpy as jnp
from jax import lax
from jax.experimental import pallas as pl
from jax.experimental.pallas import tpu as pltpu
from jax.experimental.pallas import tpu_sc as plsc

NC = 2
NS = 16
CHUNK = 128


def _chunk_size(ept):
    for ch in range(128, 0, -8):
        if ept % ch == 0:
            return ch
    return 8


def _zero_acc(zer, acc, n, s):
    rpt = n // NS
    rpt_al = ((rpt + 7) // 8) * 8
    r0 = pl.multiple_of((s * rpt) // 8 * 8, 8)
    off = 0
    while off < rpt_al:
        ln = min(CHUNK, rpt_al - off)
        pltpu.sync_copy(zer.at[pl.ds(0, ln)], acc.at[pl.ds(r0 + off, ln)])
        off += ln
    return r0, rpt_al


def _agg_body(n, nch, *refs):
    (tbl, src2, dst3, zer, out, idxs, idxd, rows0, rows1, acc,
     sem0, sem1) = refs
    ch = rows0.shape[0]

    c = lax.axis_index("c")
    s = lax.axis_index("s")
    w = s * NC + c

    r0, rpt_al = _zero_acc(zer, acc, n, s)
    plsc.subcore_barrier()

    pltpu.sync_copy(src2.at[w], idxs)
    pltpu.sync_copy(dst3.at[w], idxd)

    def gather(j, buf, sem):
        return pltpu.async_copy(tbl.at[idxs.at[pl.ds(j * ch, ch)]], buf, sem)

    def drain(j, buf, sem):
        pltpu.make_async_copy(tbl.at[idxs.at[pl.ds(j * ch, ch)]], buf,
                              sem).wait()

    gather(0, rows0, sem0)

    def step(i, _):
        a = 2 * i
        gather(a + 1, rows1, sem1)
        drain(a, rows0, sem0)
        pltpu.sync_copy(rows0, acc.at[idxd.at[a]], add=True)
        gather(a + 2, rows0, sem0)
        drain(a + 1, rows1, sem1)
        pltpu.sync_copy(rows1, acc.at[idxd.at[a + 1]], add=True)
        return 0

    assert nch % 2 == 1, "pipeline epilogue assumes odd chunk count"
    lax.fori_loop(0, (nch - 1) // 2, step, 0)
    drain(nch - 1, rows0, sem0)
    pltpu.sync_copy(rows0, acc.at[idxd.at[nch - 1]], add=True)

    plsc.subcore_barrier()
    pltpu.sync_copy(acc.at[pl.ds(r0, rpt_al)],
                    out.at[pl.ds(c * n + r0, rpt_al)])


def _deg_body(n, nch, *refs):
    (dst3, zer, ones, out, idxd, rows, acc, gsem) = refs

    c = lax.axis_index("c")
    s = lax.axis_index("s")
    w = s * NC + c

    r0, rpt_al = _zero_acc(zer, acc, n, s)
    pltpu.sync_copy(ones, rows)
    plsc.subcore_barrier()

    pltpu.sync_copy(dst3.at[w], idxd)

    def step(j, _):
        pltpu.async_copy(rows, acc.at[idxd.at[j]], gsem, add=True)
        return 0

    lax.fori_loop(0, nch, step, 0)

    def drain_step(j, _):
        pltpu.make_async_copy(rows, acc.at[idxd.at[j]], gsem).wait()
        return 0

    lax.fori_loop(0, nch, drain_step, 0)
    plsc.subcore_barrier()
    pltpu.sync_copy(acc.at[pl.ds(r0, rpt_al)],
                    out.at[pl.ds(c * n + r0, rpt_al)])


def _sc_segment_sum(table, src, dst, n):
    e = src.shape[0]
    width = table.shape[1]
    ept = e // (NC * NS)
    ch = _chunk_size(ept)
    nch = ept // ch
    src2 = src.reshape(NC * NS, ept)
    dst3 = dst.reshape(NC * NS, nch, ch)
    mesh = plsc.VectorSubcoreMesh(core_axis_name="c", subcore_axis_name="s",
                                  num_cores=NC, num_subcores=NS)
    body = functools.partial(_agg_body, n, nch)
    fn = pl.kernel(
        body,
        out_type=jax.ShapeDtypeStruct((NC * n, width), jnp.float32),
        mesh=mesh,
        scratch_types=[
            pltpu.VMEM((ept,), jnp.int32),
            pltpu.VMEM((nch, ch), jnp.int32),
            pltpu.VMEM((ch, width), jnp.float32),
            pltpu.VMEM((ch, width), jnp.float32),
            pltpu.VMEM_SHARED((n, width), jnp.float32),
            pltpu.SemaphoreType.DMA,
            pltpu.SemaphoreType.DMA,
        ],
    )
    zer = jnp.zeros((CHUNK, width), jnp.float32)
    return fn(table, src2, dst3, zer).reshape(NC, n, width)


def _sc_degree(dst, n):
    e = dst.shape[0]
    width = 128
    ept = e // (NC * NS)
    ch = _chunk_size(ept)
    nch = ept // ch
    dst3 = dst.reshape(NC * NS, nch, ch)
    mesh = plsc.VectorSubcoreMesh(core_axis_name="c", subcore_axis_name="s",
                                  num_cores=NC, num_subcores=NS)
    body = functools.partial(_deg_body, n, nch)
    fn = pl.kernel(
        body,
        out_type=jax.ShapeDtypeStruct((NC * n, width), jnp.float32),
        mesh=mesh,
        scratch_types=[
            pltpu.VMEM((nch, ch), jnp.int32),
            pltpu.VMEM((ch, width), jnp.float32),
            pltpu.VMEM_SHARED((n, width), jnp.float32),
            pltpu.SemaphoreType.DMA,
        ],
    )
    zer = jnp.zeros((CHUNK, width), jnp.float32)
    ones = jnp.ones((ch, width), jnp.float32)
    return fn(dst3, zer, ones).reshape(NC, n, width)


def _tc_prep(deg_parts, features, W1):
    n, d = features.shape
    h = W1.shape[1]

    def body(p_ref, f_ref, w_ref, q_ref, nrm_ref):
        deg = p_ref[0, :, 0:1] + p_ref[1, :, 0:1]
        nrm = lax.rsqrt(jnp.maximum(deg, 1.0))
        nrm_ref[...] = nrm
        q_ref[...] = jnp.dot(f_ref[...], w_ref[...],
                             preferred_element_type=jnp.float32) * nrm

    return pl.pallas_call(
        body,
        out_shape=(jax.ShapeDtypeStruct((n, h), jnp.float32),
                   jax.ShapeDtypeStruct((n, 1), jnp.float32)),
    )(deg_parts, features, W1)


def _tc_mid(agg_parts, norm, b1):
    n, h = agg_parts.shape[1:]

    def body(p_ref, nrm_ref, b_ref, y_ref):
        a = (p_ref[0] + p_ref[1]) * nrm_ref[...] + b_ref[...][None, :]
        x = jnp.where(a > 0, a, jnp.exp(a) - 1.0)
        y_ref[...] = x * nrm_ref[...]

    return pl.pallas_call(
        body,
        out_shape=jax.ShapeDtypeStruct((n, h), jnp.float32),
    )(agg_parts, norm, b1)


def _tc_out(agg_parts, norm, W2, b2):
    n = norm.shape[0]
    c = b2.shape[0]

    def body(p_ref, nrm_ref, w_ref, b_ref, o_ref):
        a = (p_ref[0] + p_ref[1]) * nrm_ref[...]
        o_ref[...] = jnp.dot(a, w_ref[...],
                             preferred_element_type=jnp.float32) + b_ref[...][None, :]

    return pl.pallas_call(
        body,
        out_shape=jax.ShapeDtypeStruct((n, c), jnp.float32),
    )(agg_parts, norm, W2, b2)


def kernel(edge_index, features, W1, b1, W2, b2):
    n = features.shape[0]
    src = edge_index[0]
    dst = edge_index[1]

    deg_parts = _sc_degree(dst, n)
    q, norm = _tc_prep(deg_parts, features, W1)
    agg1 = _sc_segment_sum(q, src, dst, n)
    y = _tc_mid(agg1, norm, b1)
    agg2 = _sc_segment_sum(y, src, dst, n)
    return _tc_out(agg2, norm, W2, b2)


# speedup vs baseline: 8.5930x; 1.0646x over previous
import functools

import jax
import jax.numpy as jnp
from jax import lax
from jax.experimental import pallas as pl
from jax.experimental.pallas import tpu as pltpu
from jax.experimental.pallas import tpu_sc as plsc

NC = 2
NS = 16
CHUNK = 128


def _chunk_size(ept):
    for ch in range(128, 0, -8):
        if ept % ch == 0:
            return ch
    return 8


def _zero_acc(zer, acc, n, s, g=8):
    rpt = n // NS
    rpt_al = rpt if rpt % g == 0 else ((rpt + g - 1) // g) * g
    r0 = pl.multiple_of((s * rpt) // g * g, g)
    off = 0
    while off < rpt_al:
        ln = min(CHUNK, rpt_al - off)
        pltpu.sync_copy(zer.at[pl.ds(0, ln)], acc.at[pl.ds(r0 + off, ln)])
        off += ln
    return r0, rpt_al


def _agg_body(n, nch, *refs):
    (tbl, src2, dst3, zer, out, idxs, idxd, rows0, rows1, acc,
     sem0, sem1) = refs
    ch = rows0.shape[0]

    c = lax.axis_index("c")
    s = lax.axis_index("s")
    w = s * NC + c

    r0, rpt_al = _zero_acc(zer, acc, n, s)
    plsc.subcore_barrier()

    pltpu.sync_copy(src2.at[w], idxs)
    pltpu.sync_copy(dst3.at[w], idxd)

    def gather(j, buf, sem):
        return pltpu.async_copy(tbl.at[idxs.at[pl.ds(j * ch, ch)]], buf, sem)

    def drain(j, buf, sem):
        pltpu.make_async_copy(tbl.at[idxs.at[pl.ds(j * ch, ch)]], buf,
                              sem).wait()

    gather(0, rows0, sem0)

    def step(i, _):
        a = 2 * i
        gather(a + 1, rows1, sem1)
        drain(a, rows0, sem0)
        pltpu.sync_copy(rows0, acc.at[idxd.at[a]], add=True)
        gather(a + 2, rows0, sem0)
        drain(a + 1, rows1, sem1)
        pltpu.sync_copy(rows1, acc.at[idxd.at[a + 1]], add=True)
        return 0

    assert nch % 2 == 1, "pipeline epilogue assumes odd chunk count"
    lax.fori_loop(0, (nch - 1) // 2, step, 0)
    drain(nch - 1, rows0, sem0)
    pltpu.sync_copy(rows0, acc.at[idxd.at[nch - 1]], add=True)

    plsc.subcore_barrier()
    pltpu.sync_copy(acc.at[pl.ds(r0, rpt_al)],
                    out.at[pl.ds(c * n + r0, rpt_al)])


def _deg_body(n, nch, *refs):
    (dst3, zer, ones, out, idxd, rows, acc, gsem) = refs

    c = lax.axis_index("c")
    s = lax.axis_index("s")
    w = s * NC + c

    r0, rpt_al = _zero_acc(zer, acc, n, s)
    pltpu.sync_copy(ones, rows)
    plsc.subcore_barrier()

    pltpu.sync_copy(dst3.at[w], idxd)

    def step(j, _):
        pltpu.async_copy(rows, acc.at[idxd.at[j]], gsem, add=True)
        return 0

    lax.fori_loop(0, nch, step, 0)

    def drain_step(j, _):
        pltpu.make_async_copy(rows, acc.at[idxd.at[j]], gsem).wait()
        return 0

    lax.fori_loop(0, nch, drain_step, 0)
    plsc.subcore_barrier()
    pltpu.sync_copy(acc.at[pl.ds(r0, rpt_al)],
                    out.at[pl.ds(c * n + r0, rpt_al)])


def _sc_segment_sum(table, src, dst, n):
    e = src.shape[0]
    width = table.shape[1]
    ept = e // (NC * NS)
    ch = _chunk_size(ept)
    nch = ept // ch
    src2 = src.reshape(NC * NS, ept)
    dst3 = dst.reshape(NC * NS, nch, ch)
    mesh = plsc.VectorSubcoreMesh(core_axis_name="c", subcore_axis_name="s",
                                  num_cores=NC, num_subcores=NS)
    body = functools.partial(_agg_body, n, nch)
    fn = pl.kernel(
        body,
        out_type=jax.ShapeDtypeStruct((NC * n, width), jnp.float32),
        mesh=mesh,
        scratch_types=[
            pltpu.VMEM((ept,), jnp.int32),
            pltpu.VMEM((nch, ch), jnp.int32),
            pltpu.VMEM((ch, width), jnp.float32),
            pltpu.VMEM((ch, width), jnp.float32),
            pltpu.VMEM_SHARED((n, width), jnp.float32),
            pltpu.SemaphoreType.DMA,
            pltpu.SemaphoreType.DMA,
        ],
    )
    zer = jnp.zeros((CHUNK, width), jnp.float32)
    return fn(table, src2, dst3, zer).reshape(NC, n, width)


def _sc_degree(dst, n, width=64):
    e = dst.shape[0]
    ept = e // (NC * NS)
    ch = _chunk_size(ept)
    nch = ept // ch
    dst3 = dst.reshape(NC * NS, nch, ch)
    mesh = plsc.VectorSubcoreMesh(core_axis_name="c", subcore_axis_name="s",
                                  num_cores=NC, num_subcores=NS)
    body = functools.partial(_deg_body, n, nch)
    fn = pl.kernel(
        body,
        out_type=jax.ShapeDtypeStruct((NC * n, width), jnp.float32),
        mesh=mesh,
        scratch_types=[
            pltpu.VMEM((nch, ch), jnp.int32),
            pltpu.VMEM((ch, width), jnp.float32),
            pltpu.VMEM_SHARED((n, width), jnp.float32),
            pltpu.SemaphoreType.DMA,
        ],
        compiler_params=pltpu.CompilerParams(use_tc_tiling_on_sc=False),
    )
    zer = jnp.zeros((CHUNK, width), jnp.float32)
    ones = jnp.ones((ch, width), jnp.float32)
    return fn(dst3, zer, ones).reshape(NC, n, width)


def _tc_prep(deg_parts, features, W1):
    n, d = features.shape
    h = W1.shape[1]

    def body(p_ref, f_ref, w_ref, q_ref, nrm_ref):
        deg = (p_ref[0, :, 0:1].astype(jnp.float32)
               + p_ref[1, :, 0:1].astype(jnp.float32))
        nrm = lax.rsqrt(jnp.maximum(deg, 1.0))
        nrm_ref[...] = nrm
        q_ref[...] = jnp.dot(f_ref[...], w_ref[...],
                             preferred_element_type=jnp.float32) * nrm

    return pl.pallas_call(
        body,
        out_shape=(jax.ShapeDtypeStruct((n, h), jnp.float32),
                   jax.ShapeDtypeStruct((n, 1), jnp.float32)),
    )(deg_parts, features, W1)


def _tc_mid(agg_parts, norm, b1):
    n, h = agg_parts.shape[1:]

    def body(p_ref, nrm_ref, b_ref, y_ref):
        a = (p_ref[0] + p_ref[1]) * nrm_ref[...] + b_ref[...][None, :]
        x = jnp.where(a > 0, a, jnp.exp(a) - 1.0)
        y_ref[...] = x * nrm_ref[...]

    return pl.pallas_call(
        body,
        out_shape=jax.ShapeDtypeStruct((n, h), jnp.float32),
    )(agg_parts, norm, b1)


def _tc_out(agg_parts, norm, W2, b2):
    n = norm.shape[0]
    c = b2.shape[0]

    def body(p_ref, nrm_ref, w_ref, b_ref, o_ref):
        a = (p_ref[0] + p_ref[1]) * nrm_ref[...]
        o_ref[...] = jnp.dot(a, w_ref[...],
                             preferred_element_type=jnp.float32) + b_ref[...][None, :]

    return pl.pallas_call(
        body,
        out_shape=jax.ShapeDtypeStruct((n, c), jnp.float32),
    )(agg_parts, norm, W2, b2)


def kernel(edge_index, features, W1, b1, W2, b2):
    n = features.shape[0]
    src = edge_index[0]
    dst = edge_index[1]

    deg_parts = _sc_degree(dst, n)
    q, norm = _tc_prep(deg_parts, features, W1)
    agg1 = _sc_segment_sum(q, src, dst, n)
    y = _tc_mid(agg1, norm, b1)
    agg2 = _sc_segment_sum(y, src, dst, n)
    return _tc_out(agg2, norm, W2, b2)


# speedup vs baseline: 9.2735x; 1.0792x over previous
import functools

import jax
import jax.numpy as jnp
from jax import lax
from jax.experimental import pallas as pl
from jax.experimental.pallas import tpu as pltpu
from jax.experimental.pallas import tpu_sc as plsc

NC = 2
NS = 16
CHUNK = 128


def _chunk_size(ept):
    for ch in range(128, 0, -8):
        if ept % ch == 0:
            return ch
    return 8


def _zero_acc(zer, acc, n, s, g=8):
    rpt = n // NS
    rpt_al = rpt if rpt % g == 0 else ((rpt + g - 1) // g) * g
    r0 = pl.multiple_of((s * rpt) // g * g, g)
    off = 0
    while off < rpt_al:
        ln = min(CHUNK, rpt_al - off)
        pltpu.sync_copy(zer.at[pl.ds(0, ln)], acc.at[pl.ds(r0 + off, ln)])
        off += ln
    return r0, rpt_al


def _agg_body(n, nch, *refs):
    (tbl, src2, dst3, zer, out, idxs, idxd, rows0, rows1, acc,
     sem0, sem1) = refs
    ch = rows0.shape[0]

    c = lax.axis_index("c")
    s = lax.axis_index("s")
    w = s * NC + c

    r0, rpt_al = _zero_acc(zer, acc, n, s)
    plsc.subcore_barrier()

    pltpu.sync_copy(src2.at[w], idxs)
    pltpu.sync_copy(dst3.at[w], idxd)

    def gather(j, buf, sem):
        return pltpu.async_copy(tbl.at[idxs.at[pl.ds(j * ch, ch)]], buf, sem)

    def drain(j, buf, sem):
        pltpu.make_async_copy(tbl.at[idxs.at[pl.ds(j * ch, ch)]], buf,
                              sem).wait()

    gather(0, rows0, sem0)

    def step(i, _):
        a = 2 * i
        gather(a + 1, rows1, sem1)
        drain(a, rows0, sem0)
        pltpu.sync_copy(rows0, acc.at[idxd.at[a]], add=True)
        gather(a + 2, rows0, sem0)
        drain(a + 1, rows1, sem1)
        pltpu.sync_copy(rows1, acc.at[idxd.at[a + 1]], add=True)
        return 0

    assert nch % 2 == 1, "pipeline epilogue assumes odd chunk count"
    lax.fori_loop(0, (nch - 1) // 2, step, 0)
    drain(nch - 1, rows0, sem0)
    pltpu.sync_copy(rows0, acc.at[idxd.at[nch - 1]], add=True)

    plsc.subcore_barrier()
    pltpu.sync_copy(acc.at[pl.ds(r0, rpt_al)],
                    out.at[pl.ds(c * n + r0, rpt_al)])


def _deg_body(n, nch, *refs):
    (dst3, zer, ones, out, idxd, rows, acc, gsem) = refs

    c = lax.axis_index("c")
    s = lax.axis_index("s")
    w = s * NC + c

    r0, rpt_al = _zero_acc(zer, acc, n, s)
    pltpu.sync_copy(ones, rows)
    plsc.subcore_barrier()

    pltpu.sync_copy(dst3.at[w], idxd)

    def step(j, _):
        pltpu.async_copy(rows, acc.at[idxd.at[j]], gsem, add=True)
        return 0

    lax.fori_loop(0, nch, step, 0)

    def drain_step(j, _):
        pltpu.make_async_copy(rows, acc.at[idxd.at[j]], gsem).wait()
        return 0

    lax.fori_loop(0, nch, drain_step, 0)
    plsc.subcore_barrier()
    pltpu.sync_copy(acc.at[pl.ds(r0, rpt_al)],
                    out.at[pl.ds(c * n + r0, rpt_al)])


def _sc_segment_sum(table, src, dst, n):
    e = src.shape[0]
    width = table.shape[1]
    ept = e // (NC * NS)
    ch = _chunk_size(ept)
    nch = ept // ch
    src2 = src.reshape(NC * NS, ept)
    dst3 = dst.reshape(NC * NS, nch, ch)
    mesh = plsc.VectorSubcoreMesh(core_axis_name="c", subcore_axis_name="s",
                                  num_cores=NC, num_subcores=NS)
    body = functools.partial(_agg_body, n, nch)
    fn = pl.kernel(
        body,
        out_type=jax.ShapeDtypeStruct((NC * n, width), jnp.float32),
        mesh=mesh,
        scratch_types=[
            pltpu.VMEM((ept,), jnp.int32),
            pltpu.VMEM((nch, ch), jnp.int32),
            pltpu.VMEM((ch, width), jnp.float32),
            pltpu.VMEM((ch, width), jnp.float32),
            pltpu.VMEM_SHARED((n, width), jnp.float32),
            pltpu.SemaphoreType.DMA,
            pltpu.SemaphoreType.DMA,
        ],
        compiler_params=(None if width == 128 else
                         pltpu.CompilerParams(use_tc_tiling_on_sc=False)),
    )
    zer = jnp.zeros((CHUNK, width), jnp.float32)
    return fn(table, src2, dst3, zer).reshape(NC, n, width)


def _sc_degree(dst, n, width=64):
    e = dst.shape[0]
    ept = e // (NC * NS)
    ch = _chunk_size(ept)
    nch = ept // ch
    dst3 = dst.reshape(NC * NS, nch, ch)
    mesh = plsc.VectorSubcoreMesh(core_axis_name="c", subcore_axis_name="s",
                                  num_cores=NC, num_subcores=NS)
    body = functools.partial(_deg_body, n, nch)
    fn = pl.kernel(
        body,
        out_type=jax.ShapeDtypeStruct((NC * n, width), jnp.float32),
        mesh=mesh,
        scratch_types=[
            pltpu.VMEM((nch, ch), jnp.int32),
            pltpu.VMEM((ch, width), jnp.float32),
            pltpu.VMEM_SHARED((n, width), jnp.float32),
            pltpu.SemaphoreType.DMA,
        ],
        compiler_params=pltpu.CompilerParams(use_tc_tiling_on_sc=False),
    )
    zer = jnp.zeros((CHUNK, width), jnp.float32)
    ones = jnp.ones((ch, width), jnp.float32)
    return fn(dst3, zer, ones).reshape(NC, n, width)


def _tc_prep(deg_parts, features, W1):
    n, d = features.shape
    h = W1.shape[1]

    def body(p_ref, f_ref, w_ref, q_ref, nrm_ref):
        deg = (p_ref[0, :, 0:1].astype(jnp.float32)
               + p_ref[1, :, 0:1].astype(jnp.float32))
        nrm = lax.rsqrt(jnp.maximum(deg, 1.0))
        nrm_ref[...] = nrm
        q_ref[...] = jnp.dot(f_ref[...], w_ref[...],
                             preferred_element_type=jnp.float32) * nrm

    return pl.pallas_call(
        body,
        out_shape=(jax.ShapeDtypeStruct((n, h), jnp.float32),
                   jax.ShapeDtypeStruct((n, 1), jnp.float32)),
    )(deg_parts, features, W1)


def _tc_mid(agg_parts, norm, b1, W2):
    n = norm.shape[0]
    c = W2.shape[1]

    def body(p_ref, nrm_ref, b_ref, w_ref, z_ref):
        a = (p_ref[0] + p_ref[1]) * nrm_ref[...] + b_ref[...][None, :]
        x = jnp.where(a > 0, a, jnp.exp(a) - 1.0)
        z_ref[...] = jnp.dot(x, w_ref[...],
                             preferred_element_type=jnp.float32) * nrm_ref[...]

    return pl.pallas_call(
        body,
        out_shape=jax.ShapeDtypeStruct((n, c), jnp.float32),
    )(agg_parts, norm, b1, W2)


def _tc_out(agg_parts, norm, b2):
    n = norm.shape[0]
    c = b2.shape[0]

    def body(p_ref, nrm_ref, b_ref, o_ref):
        o_ref[...] = (p_ref[0] + p_ref[1]) * nrm_ref[...] + b_ref[...][None, :]

    return pl.pallas_call(
        body,
        out_shape=jax.ShapeDtypeStruct((n, c), jnp.float32),
    )(agg_parts, norm, b2)


def kernel(edge_index, features, W1, b1, W2, b2):
    n = features.shape[0]
    src = edge_index[0]
    dst = edge_index[1]

    deg_parts = _sc_degree(dst, n)
    q, norm = _tc_prep(deg_parts, features, W1)
    agg1 = _sc_segment_sum(q, src, dst, n)
    z = _tc_mid(agg1, norm, b1, W2)
    agg2 = _sc_segment_sum(z, src, dst, n)
    return _tc_out(agg2, norm, b2)


# speedup vs baseline: 10.3041x; 1.1111x over previous
import functools

import jax
import jax.numpy as jnp
from jax import lax
from jax.experimental import pallas as pl
from jax.experimental.pallas import tpu as pltpu
from jax.experimental.pallas import tpu_sc as plsc

NC = 2
NS = 16
CHUNK = 128


def _chunk_size(ept):
    for ch in range(128, 0, -8):
        if ept % ch == 0:
            return ch
    return 8


def _zero_acc(zer, acc, n, s, g=8):
    rpt = n // NS
    rpt_al = rpt if rpt % g == 0 else ((rpt + g - 1) // g) * g
    r0 = pl.multiple_of((s * rpt) // g * g, g)
    off = 0
    while off < rpt_al:
        ln = min(CHUNK, rpt_al - off)
        pltpu.sync_copy(zer.at[pl.ds(0, ln)], acc.at[pl.ds(r0 + off, ln)])
        off += ln
    return r0, rpt_al


def _agg_body(n, nch, depth, *refs):
    tbl, src2, dst3, zer, out, idxs, idxd = refs[:7]
    bufs = refs[7:7 + depth]
    acc = refs[7 + depth]
    sems = refs[8 + depth:8 + 2 * depth]
    ch = bufs[0].shape[0]

    c = lax.axis_index("c")
    s = lax.axis_index("s")
    w = s * NC + c

    r0, rpt_al = _zero_acc(zer, acc, n, s)
    plsc.subcore_barrier()

    pltpu.sync_copy(src2.at[w], idxs)
    pltpu.sync_copy(dst3.at[w], idxd)

    def gather(j, t):
        return pltpu.async_copy(tbl.at[idxs.at[pl.ds(j * ch, ch)]],
                                bufs[t], sems[t])

    def drain(j, t):
        pltpu.make_async_copy(tbl.at[idxs.at[pl.ds(j * ch, ch)]],
                              bufs[t], sems[t]).wait()

    for t in range(depth):
        if t < nch:
            gather(t, t)

    def step(i, _):
        base = depth * i
        for t in range(depth):
            j = base + t
            drain(j, t)
            pltpu.sync_copy(bufs[t], acc.at[idxd.at[j]], add=True)

            @pl.when(j + depth < nch)
            def _():
                gather(j + depth, t)
        return 0

    lax.fori_loop(0, nch // depth, step, 0)
    for j in range(nch - nch % depth, nch):
        t = j % depth
        drain(j, t)
        pltpu.sync_copy(bufs[t], acc.at[idxd.at[j]], add=True)

    plsc.subcore_barrier()
    pltpu.sync_copy(acc.at[pl.ds(r0, rpt_al)],
                    out.at[pl.ds(c * n + r0, rpt_al)])


def _deg_body(n, nch, *refs):
    (dst3, zer, ones, out, idxd, rows, acc, gsem) = refs

    c = lax.axis_index("c")
    s = lax.axis_index("s")
    w = s * NC + c

    r0, rpt_al = _zero_acc(zer, acc, n, s)
    pltpu.sync_copy(ones, rows)
    plsc.subcore_barrier()

    pltpu.sync_copy(dst3.at[w], idxd)

    def step(j, _):
        pltpu.async_copy(rows, acc.at[idxd.at[j]], gsem, add=True)
        return 0

    lax.fori_loop(0, nch, step, 0)

    def drain_step(j, _):
        pltpu.make_async_copy(rows, acc.at[idxd.at[j]], gsem).wait()
        return 0

    lax.fori_loop(0, nch, drain_step, 0)
    plsc.subcore_barrier()
    pltpu.sync_copy(acc.at[pl.ds(r0, rpt_al)],
                    out.at[pl.ds(c * n + r0, rpt_al)])


def _sc_segment_sum(table, src, dst, n):
    e = src.shape[0]
    width = table.shape[1]
    ept = e // (NC * NS)
    ch = _chunk_size(ept)
    nch = ept // ch
    src2 = src.reshape(NC * NS, ept)
    dst3 = dst.reshape(NC * NS, nch, ch)
    mesh = plsc.VectorSubcoreMesh(core_axis_name="c", subcore_axis_name="s",
                                  num_cores=NC, num_subcores=NS)
    depth = 2 if width == 128 else 3
    body = functools.partial(_agg_body, n, nch, depth)
    fn = pl.kernel(
        body,
        out_type=jax.ShapeDtypeStruct((NC * n, width), jnp.float32),
        mesh=mesh,
        scratch_types=(
            [pltpu.VMEM((ept,), jnp.int32),
             pltpu.VMEM((nch, ch), jnp.int32)]
            + [pltpu.VMEM((ch, width), jnp.float32) for _ in range(depth)]
            + [pltpu.VMEM_SHARED((n, width), jnp.float32)]
            + [pltpu.SemaphoreType.DMA for _ in range(depth)]
        ),
        compiler_params=(None if width == 128 else
                         pltpu.CompilerParams(use_tc_tiling_on_sc=False)),
    )
    zer = jnp.zeros((CHUNK, width), jnp.float32)
    return fn(table, src2, dst3, zer).reshape(NC, n, width)


def _sc_degree(dst, n, width=32):
    e = dst.shape[0]
    ept = e // (NC * NS)
    ch = _chunk_size(ept)
    nch = ept // ch
    dst3 = dst.reshape(NC * NS, nch, ch)
    mesh = plsc.VectorSubcoreMesh(core_axis_name="c", subcore_axis_name="s",
                                  num_cores=NC, num_subcores=NS)
    body = functools.partial(_deg_body, n, nch)
    fn = pl.kernel(
        body,
        out_type=jax.ShapeDtypeStruct((NC * n, width), jnp.float32),
        mesh=mesh,
        scratch_types=[
            pltpu.VMEM((nch, ch), jnp.int32),
            pltpu.VMEM((ch, width), jnp.float32),
            pltpu.VMEM_SHARED((n, width), jnp.float32),
            pltpu.SemaphoreType.DMA,
        ],
        compiler_params=pltpu.CompilerParams(use_tc_tiling_on_sc=False),
    )
    zer = jnp.zeros((CHUNK, width), jnp.float32)
    ones = jnp.ones((ch, width), jnp.float32)
    return fn(dst3, zer, ones).reshape(NC, n, width)


def _tc_prep(deg_parts, features, W1):
    n, d = features.shape
    h = W1.shape[1]

    def body(p_ref, f_ref, w_ref, q_ref, nrm_ref):
        deg = (p_ref[0, :, 0:1].astype(jnp.float32)
               + p_ref[1, :, 0:1].astype(jnp.float32))
        nrm = lax.rsqrt(jnp.maximum(deg, 1.0))
        nrm_ref[...] = nrm
        q_ref[...] = jnp.dot(f_ref[...], w_ref[...],
                             preferred_element_type=jnp.float32) * nrm

    return pl.pallas_call(
        body,
        out_shape=(jax.ShapeDtypeStruct((n, h), jnp.float32),
                   jax.ShapeDtypeStruct((n, 1), jnp.float32)),
    )(deg_parts, features, W1)


def _tc_mid(agg_parts, norm, b1, W2):
    n = norm.shape[0]
    c = W2.shape[1]

    def body(p_ref, nrm_ref, b_ref, w_ref, z_ref):
        a = (p_ref[0] + p_ref[1]) * nrm_ref[...] + b_ref[...][None, :]
        x = jnp.where(a > 0, a, jnp.exp(a) - 1.0)
        z_ref[...] = jnp.dot(x, w_ref[...],
                             preferred_element_type=jnp.float32) * nrm_ref[...]

    return pl.pallas_call(
        body,
        out_shape=jax.ShapeDtypeStruct((n, c), jnp.float32),
    )(agg_parts, norm, b1, W2)


def _tc_out(agg_parts, norm, b2):
    n = norm.shape[0]
    c = b2.shape[0]

    def body(p_ref, nrm_ref, b_ref, o_ref):
        o_ref[...] = (p_ref[0] + p_ref[1]) * nrm_ref[...] + b_ref[...][None, :]

    return pl.pallas_call(
        body,
        out_shape=jax.ShapeDtypeStruct((n, c), jnp.float32),
    )(agg_parts, norm, b2)


def kernel(edge_index, features, W1, b1, W2, b2):
    n = features.shape[0]
    src = edge_index[0]
    dst = edge_index[1]

    deg_parts = _sc_degree(dst, n)
    q, norm = _tc_prep(deg_parts, features, W1)
    agg1 = _sc_segment_sum(q, src, dst, n)
    z = _tc_mid(agg1, norm, b1, W2)
    agg2 = _sc_segment_sum(z, src, dst, n)
    return _tc_out(agg2, norm, b2)


# speedup vs baseline: 11.1225x; 1.0794x over previous
import functools

import jax
import jax.numpy as jnp
from jax import lax
from jax.experimental import pallas as pl
from jax.experimental.pallas import tpu as pltpu
from jax.experimental.pallas import tpu_sc as plsc

NC = 2
NS = 16
CHUNK = 128


def _chunk_size(ept):
    for ch in range(128, 0, -8):
        if ept % ch == 0:
            return ch
    return 8


def _zero_acc(zer, acc, n, s, g=8):
    rpt = n // NS
    rpt_al = rpt if rpt % g == 0 else ((rpt + g - 1) // g) * g
    r0 = pl.multiple_of((s * rpt) // g * g, g)
    off = 0
    while off < rpt_al:
        ln = min(CHUNK, rpt_al - off)
        pltpu.sync_copy(zer.at[pl.ds(0, ln)], acc.at[pl.ds(r0 + off, ln)])
        off += ln
    return r0, rpt_al


def _agg_body(n, nch, depth, *refs):
    tbl, src2, dst2, zer, out, idxs, idxd = refs[:7]
    bufs = refs[7:7 + depth]
    acc = refs[7 + depth]
    sems = refs[8 + depth:8 + 2 * depth]
    ch = bufs[0].shape[0]

    c = lax.axis_index("c")
    s = lax.axis_index("s")
    w = s * NC + c

    r0, rpt_al = _zero_acc(zer, acc, n, s)
    plsc.subcore_barrier()

    pltpu.sync_copy(src2.at[w], idxs)
    pltpu.sync_copy(dst2.at[w], idxd)

    def gather(j, t):
        return pltpu.async_copy(tbl.at[idxs.at[pl.ds(j * ch, ch)]],
                                bufs[t], sems[t])

    def drain(j, t):
        pltpu.make_async_copy(tbl.at[idxs.at[pl.ds(j * ch, ch)]],
                              bufs[t], sems[t]).wait()

    for t in range(depth):
        if t < nch:
            gather(t, t)

    def step(i, _):
        base = depth * i
        for t in range(depth):
            j = base + t
            drain(j, t)
            pltpu.sync_copy(bufs[t], acc.at[idxd.at[pl.ds(j * ch, ch)]],
                            add=True)

            @pl.when(j + depth < nch)
            def _():
                gather(j + depth, t)
        return 0

    lax.fori_loop(0, nch // depth, step, 0)
    for j in range(nch - nch % depth, nch):
        t = j % depth
        drain(j, t)
        pltpu.sync_copy(bufs[t], acc.at[idxd.at[pl.ds(j * ch, ch)]],
                        add=True)

    plsc.subcore_barrier()
    pltpu.sync_copy(acc.at[pl.ds(r0, rpt_al)],
                    out.at[pl.ds(c * n + r0, rpt_al)])


def _deg_body(n, nch, *refs):
    (dst3, zer, ones, out, idxd, rows, acc, gsem) = refs

    c = lax.axis_index("c")
    s = lax.axis_index("s")
    w = s * NC + c

    r0, rpt_al = _zero_acc(zer, acc, n, s)
    pltpu.sync_copy(ones, rows)
    plsc.subcore_barrier()

    pltpu.sync_copy(dst3.at[w], idxd)

    def step(j, _):
        pltpu.async_copy(rows, acc.at[idxd.at[j]], gsem, add=True)
        return 0

    lax.fori_loop(0, nch, step, 0)

    def drain_step(j, _):
        pltpu.make_async_copy(rows, acc.at[idxd.at[j]], gsem).wait()
        return 0

    lax.fori_loop(0, nch, drain_step, 0)
    plsc.subcore_barrier()
    pltpu.sync_copy(acc.at[pl.ds(r0, rpt_al)],
                    out.at[pl.ds(c * n + r0, rpt_al)])


def _sc_segment_sum(table, src, dst, n):
    e = src.shape[0]
    width = table.shape[1]
    ept = e // (NC * NS)
    ch = _chunk_size(ept)
    nch = ept // ch
    src2 = src.reshape(NC * NS, ept)
    dst2 = dst.reshape(NC * NS, ept)
    mesh = plsc.VectorSubcoreMesh(core_axis_name="c", subcore_axis_name="s",
                                  num_cores=NC, num_subcores=NS)
    depth = 3
    body = functools.partial(_agg_body, n, nch, depth)
    fn = pl.kernel(
        body,
        out_type=jax.ShapeDtypeStruct((NC * n, width), jnp.float32),
        mesh=mesh,
        scratch_types=(
            [pltpu.VMEM((ept,), jnp.int32),
             pltpu.VMEM((ept,), jnp.int32)]
            + [pltpu.VMEM((ch, width), jnp.float32) for _ in range(depth)]
            + [pltpu.VMEM_SHARED((n, width), jnp.float32)]
            + [pltpu.SemaphoreType.DMA for _ in range(depth)]
        ),
        compiler_params=(None if width == 128 else
                         pltpu.CompilerParams(use_tc_tiling_on_sc=False)),
    )
    zer = jnp.zeros((CHUNK, width), jnp.float32)
    return fn(table, src2, dst2, zer).reshape(NC, n, width)


def _sc_degree(dst, n, width=32):
    e = dst.shape[0]
    ept = e // (NC * NS)
    ch = _chunk_size(ept)
    nch = ept // ch
    dst3 = dst.reshape(NC * NS, nch, ch)
    mesh = plsc.VectorSubcoreMesh(core_axis_name="c", subcore_axis_name="s",
                                  num_cores=NC, num_subcores=NS)
    body = functools.partial(_deg_body, n, nch)
    fn = pl.kernel(
        body,
        out_type=jax.ShapeDtypeStruct((NC * n, width), jnp.float32),
        mesh=mesh,
        scratch_types=[
            pltpu.VMEM((nch, ch), jnp.int32),
            pltpu.VMEM((ch, width), jnp.float32),
            pltpu.VMEM_SHARED((n, width), jnp.float32),
            pltpu.SemaphoreType.DMA,
        ],
        compiler_params=pltpu.CompilerParams(use_tc_tiling_on_sc=False),
    )
    zer = jnp.zeros((CHUNK, width), jnp.float32)
    ones = jnp.ones((ch, width), jnp.float32)
    return fn(dst3, zer, ones).reshape(NC, n, width)


def _tc_prep(deg_parts, features, W1):
    n, d = features.shape
    h = W1.shape[1]

    def body(p_ref, f_ref, w_ref, q_ref, nrm_ref):
        deg = (p_ref[0, :, 0:1].astype(jnp.float32)
               + p_ref[1, :, 0:1].astype(jnp.float32))
        nrm = lax.rsqrt(jnp.maximum(deg, 1.0))
        nrm_ref[...] = nrm
        q_ref[...] = jnp.dot(f_ref[...], w_ref[...],
                             preferred_element_type=jnp.float32) * nrm

    return pl.pallas_call(
        body,
        out_shape=(jax.ShapeDtypeStruct((n, h), jnp.float32),
                   jax.ShapeDtypeStruct((n, 1), jnp.float32)),
    )(deg_parts, features, W1)


def _tc_mid(agg_parts, norm, b1, W2):
    n = norm.shape[0]
    c = W2.shape[1]

    def body(p_ref, nrm_ref, b_ref, w_ref, z_ref):
        a = (p_ref[0] + p_ref[1]) * nrm_ref[...] + b_ref[...][None, :]
        x = jnp.where(a > 0, a, jnp.exp(a) - 1.0)
        z_ref[...] = jnp.dot(x, w_ref[...],
                             preferred_element_type=jnp.float32) * nrm_ref[...]

    return pl.pallas_call(
        body,
        out_shape=jax.ShapeDtypeStruct((n, c), jnp.float32),
    )(agg_parts, norm, b1, W2)


def _tc_out(agg_parts, norm, b2):
    n = norm.shape[0]
    c = b2.shape[0]

    def body(p_ref, nrm_ref, b_ref, o_ref):
        o_ref[...] = (p_ref[0] + p_ref[1]) * nrm_ref[...] + b_ref[...][None, :]

    return pl.pallas_call(
        body,
        out_shape=jax.ShapeDtypeStruct((n, c), jnp.float32),
    )(agg_parts, norm, b2)


def kernel(edge_index, features, W1, b1, W2, b2):
    n = features.shape[0]
    src = edge_index[0]
    dst = edge_index[1]

    deg_parts = _sc_degree(dst, n)
    q, norm = _tc_prep(deg_parts, features, W1)
    agg1 = _sc_segment_sum(q, src, dst, n)
    z = _tc_mid(agg1, norm, b1, W2)
    agg2 = _sc_segment_sum(z, src, dst, n)
    return _tc_out(agg2, norm, b2)


# speedup vs baseline: 11.5286x; 1.0365x over previous
import functools

import jax
import jax.numpy as jnp
from jax import lax
from jax.experimental import pallas as pl
from jax.experimental.pallas import tpu as pltpu
from jax.experimental.pallas import tpu_sc as plsc

NC = 2
NS = 16
CHUNK = 128


def _chunk_size(ept):
    for ch in range(128, 0, -8):
        if ept % ch == 0:
            return ch
    return 8


def _zero_acc(zer, acc, n, s, g=8):
    rpt = n // NS
    rpt_al = rpt if rpt % g == 0 else ((rpt + g - 1) // g) * g
    r0 = pl.multiple_of((s * rpt) // g * g, g)
    off = 0
    while off < rpt_al:
        ln = min(CHUNK, rpt_al - off)
        pltpu.sync_copy(zer.at[pl.ds(0, ln)], acc.at[pl.ds(r0 + off, ln)])
        off += ln
    return r0, rpt_al


def _agg_body(n, nch, depth, *refs):
    tbl, src2, dst2, zer, out, idxs, idxd = refs[:7]
    bufs = refs[7:7 + depth]
    acc = refs[7 + depth]
    sems = refs[8 + depth:8 + 2 * depth]
    ch = bufs[0].shape[0]

    c = lax.axis_index("c")
    s = lax.axis_index("s")
    w = s * NC + c

    r0, rpt_al = _zero_acc(zer, acc, n, s)
    plsc.subcore_barrier()

    pltpu.sync_copy(src2.at[w], idxs)
    pltpu.sync_copy(dst2.at[w], idxd)

    def gather(j, t):
        return pltpu.async_copy(tbl.at[idxs.at[pl.ds(j * ch, ch)]],
                                bufs[t], sems[t])

    def drain(j, t):
        pltpu.make_async_copy(tbl.at[idxs.at[pl.ds(j * ch, ch)]],
                              bufs[t], sems[t]).wait()

    for t in range(depth):
        if t < nch:
            gather(t, t)

    def step(i, _):
        base = depth * i
        for t in range(depth):
            j = base + t
            drain(j, t)
            pltpu.sync_copy(bufs[t], acc.at[idxd.at[pl.ds(j * ch, ch)]],
                            add=True)

            @pl.when(j + depth < nch)
            def _():
                gather(j + depth, t)
        return 0

    lax.fori_loop(0, nch // depth, step, 0)
    for j in range(nch - nch % depth, nch):
        t = j % depth
        drain(j, t)
        pltpu.sync_copy(bufs[t], acc.at[idxd.at[pl.ds(j * ch, ch)]],
                        add=True)

    plsc.subcore_barrier()
    pltpu.sync_copy(acc.at[pl.ds(r0, rpt_al)],
                    out.at[pl.ds(c * n + r0, rpt_al)])


def _deg_body(n, nch, *refs):
    (dst3, zer, ones, out, idxd, rows, acc, gsem) = refs

    c = lax.axis_index("c")
    s = lax.axis_index("s")
    w = s * NC + c

    r0, rpt_al = _zero_acc(zer, acc, n, s)
    pltpu.sync_copy(ones, rows)
    plsc.subcore_barrier()

    pltpu.sync_copy(dst3.at[w], idxd)

    def step(j, _):
        pltpu.async_copy(rows, acc.at[idxd.at[j]], gsem, add=True)
        return 0

    lax.fori_loop(0, nch, step, 0)

    def drain_step(j, _):
        pltpu.make_async_copy(rows, acc.at[idxd.at[j]], gsem).wait()
        return 0

    lax.fori_loop(0, nch, drain_step, 0)
    plsc.subcore_barrier()
    pltpu.sync_copy(acc.at[pl.ds(r0, rpt_al)],
                    out.at[pl.ds(c * n + r0, rpt_al)])


def _sc_segment_sum(table, src, dst, n):
    e = src.shape[0]
    width = table.shape[1]
    ept = e // (NC * NS)
    ch = _chunk_size(ept)
    nch = ept // ch
    src2 = src.reshape(NC * NS, ept)
    dst2 = dst.reshape(NC * NS, ept)
    mesh = plsc.VectorSubcoreMesh(core_axis_name="c", subcore_axis_name="s",
                                  num_cores=NC, num_subcores=NS)
    depth = 3 if width == 128 else 5
    body = functools.partial(_agg_body, n, nch, depth)
    fn = pl.kernel(
        body,
        out_type=jax.ShapeDtypeStruct((NC * n, width), jnp.float32),
        mesh=mesh,
        scratch_types=(
            [pltpu.VMEM((ept,), jnp.int32),
             pltpu.VMEM((ept,), jnp.int32)]
            + [pltpu.VMEM((ch, width), jnp.float32) for _ in range(depth)]
            + [pltpu.VMEM_SHARED((n, width), jnp.float32)]
            + [pltpu.SemaphoreType.DMA for _ in range(depth)]
        ),
        compiler_params=(None if width == 128 else
                         pltpu.CompilerParams(use_tc_tiling_on_sc=False)),
    )
    zer = jnp.zeros((CHUNK, width), jnp.float32)
    return fn(table, src2, dst2, zer).reshape(NC, n, width)


def _sc_degree(dst, n, width=32):
    e = dst.shape[0]
    ept = e // (NC * NS)
    ch = _chunk_size(ept)
    nch = ept // ch
    dst3 = dst.reshape(NC * NS, nch, ch)
    mesh = plsc.VectorSubcoreMesh(core_axis_name="c", subcore_axis_name="s",
                                  num_cores=NC, num_subcores=NS)
    body = functools.partial(_deg_body, n, nch)
    fn = pl.kernel(
        body,
        out_type=jax.ShapeDtypeStruct((NC * n, width), jnp.float32),
        mesh=mesh,
        scratch_types=[
            pltpu.VMEM((nch, ch), jnp.int32),
            pltpu.VMEM((ch, width), jnp.float32),
            pltpu.VMEM_SHARED((n, width), jnp.float32),
            pltpu.SemaphoreType.DMA,
        ],
        compiler_params=pltpu.CompilerParams(use_tc_tiling_on_sc=False),
    )
    zer = jnp.zeros((CHUNK, width), jnp.float32)
    ones = jnp.ones((ch, width), jnp.float32)
    return fn(dst3, zer, ones).reshape(NC, n, width)


def _tc_prep(deg_parts, features, W1):
    n, d = features.shape
    h = W1.shape[1]

    def body(p_ref, f_ref, w_ref, q_ref, nrm_ref):
        deg = (p_ref[0, :, 0:1].astype(jnp.float32)
               + p_ref[1, :, 0:1].astype(jnp.float32))
        nrm = lax.rsqrt(jnp.maximum(deg, 1.0))
        nrm_ref[...] = nrm
        q_ref[...] = jnp.dot(f_ref[...], w_ref[...],
                             preferred_element_type=jnp.float32) * nrm

    return pl.pallas_call(
        body,
        out_shape=(jax.ShapeDtypeStruct((n, h), jnp.float32),
                   jax.ShapeDtypeStruct((n, 1), jnp.float32)),
    )(deg_parts, features, W1)


def _tc_mid(agg_parts, norm, b1, W2):
    n = norm.shape[0]
    c = W2.shape[1]

    def body(p_ref, nrm_ref, b_ref, w_ref, z_ref):
        a = (p_ref[0] + p_ref[1]) * nrm_ref[...] + b_ref[...][None, :]
        x = jnp.where(a > 0, a, jnp.exp(a) - 1.0)
        z_ref[...] = jnp.dot(x, w_ref[...],
                             preferred_element_type=jnp.float32) * nrm_ref[...]

    return pl.pallas_call(
        body,
        out_shape=jax.ShapeDtypeStruct((n, c), jnp.float32),
    )(agg_parts, norm, b1, W2)


def _tc_out(agg_parts, norm, b2):
    n = norm.shape[0]
    c = b2.shape[0]

    def body(p_ref, nrm_ref, b_ref, o_ref):
        o_ref[...] = (p_ref[0] + p_ref[1]) * nrm_ref[...] + b_ref[...][None, :]

    return pl.pallas_call(
        body,
        out_shape=jax.ShapeDtypeStruct((n, c), jnp.float32),
    )(agg_parts, norm, b2)


def kernel(edge_index, features, W1, b1, W2, b2):
    n = features.shape[0]
    src = edge_index[0]
    dst = edge_index[1]

    deg_parts = _sc_degree(dst, n)
    q, norm = _tc_prep(deg_parts, features, W1)
    agg1 = _sc_segment_sum(q, src, dst, n)
    z = _tc_mid(agg1, norm, b1, W2)
    agg2 = _sc_segment_sum(z, src, dst, n)
    return _tc_out(agg2, norm, b2)
